# baseline (speedup 1.0000x reference)
import functools

import jax
import jax.numpy as jnp
from jax import lax
from jax.experimental import pallas as pl
from jax.experimental.pallas import tpu as pltpu
from jax.experimental.pallas import tpu_sc as plsc

NW = 32
NC = 2


def _sc_body(per_w, nacc, x_hbm, edge_hbm,
             s_out, deg_out, x_v, src_v, dst_v, s_part, deg_part,
             sbuf, dbuf, os_v, od_v, sem, s_all, deg_all):
    c = lax.axis_index("c")
    s = lax.axis_index("s")
    wid = s * NC + c
    cp1 = pltpu.async_copy(x_hbm, x_v, sem)
    cp2 = pltpu.async_copy(edge_hbm.at[0, pl.ds(wid * per_w, per_w)], src_v, sem)
    cp3 = pltpu.async_copy(edge_hbm.at[1, pl.ds(wid * per_w, per_w)], dst_v, sem)

    zeros16 = jnp.zeros((16,), jnp.float32)

    @plsc.parallel_loop(0, nacc // 16, unroll=8)
    def _zero(j):
        sl = pl.ds(j * 16, 16)
        s_part[sl] = zeros16
        deg_part[sl] = zeros16

    cp1.wait()
    cp2.wait()
    cp3.wait()

    ones16 = jnp.ones((16,), jnp.float32)

    def one_vreg(j):
        base = j * 16
        idx_s = src_v[pl.ds(base, 16)]
        idx_d = dst_v[pl.ds(base, 16)]
        vals = plsc.load_gather(x_v, [idx_s])
        plsc.addupdate_scatter(s_part, [idx_d], vals)
        plsc.addupdate_scatter(deg_part, [idx_d], ones16)

    plsc.parallel_loop(0, per_w // 16, unroll=8)(one_vreg)

    pltpu.sync_copy(s_part, s_all.at[s])
    pltpu.sync_copy(deg_part, deg_all.at[s])
    plsc.subcore_barrier()

    w = nacc // 16
    pltpu.sync_copy(s_all.at[:, pl.ds(s * w, w)], sbuf)
    pltpu.sync_copy(deg_all.at[:, pl.ds(s * w, w)], dbuf)

    @plsc.parallel_loop(0, w // 16, unroll=2)
    def _red(v):
        sl = pl.ds(v * 16, 16)
        accs = sbuf[0, sl]
        accd = dbuf[0, sl]
        for k in range(1, 16):
            accs = accs + sbuf[k, sl]
            accd = accd + dbuf[k, sl]
        os_v[sl] = accs
        od_v[sl] = accd
    pltpu.sync_copy(os_v, s_out.at[c, pl.ds(s * w, w)])
    pltpu.sync_copy(od_v, deg_out.at[c, pl.ds(s * w, w)])


def _segment_sums(x_flat, edge_index, n):
    e = edge_index.shape[1]
    per_w = e // NW
    nacc = -(-n // 256) * 256
    w = nacc // 16

    fn = pl.kernel(
        functools.partial(_sc_body, per_w, nacc),
        out_type=[jax.ShapeDtypeStruct((NC, nacc), jnp.float32),
                  jax.ShapeDtypeStruct((NC, nacc), jnp.float32)],
        mesh=plsc.VectorSubcoreMesh(core_axis_name="c", subcore_axis_name="s"),
        compiler_params=pltpu.CompilerParams(needs_layout_passes=False,
                                             use_tc_tiling_on_sc=False,
                                             skip_device_barrier=True),
        scratch_types=[
            pltpu.VMEM((x_flat.shape[0],), jnp.float32),
            pltpu.VMEM((per_w,), jnp.int32),
            pltpu.VMEM((per_w,), jnp.int32),
            pltpu.VMEM((nacc,), jnp.float32),
            pltpu.VMEM((nacc,), jnp.float32),
            pltpu.VMEM((16, w), jnp.float32),
            pltpu.VMEM((16, w), jnp.float32),
            pltpu.VMEM((w,), jnp.float32),
            pltpu.VMEM((w,), jnp.float32),
            pltpu.SemaphoreType.DMA,
            pltpu.VMEM_SHARED((16, nacc), jnp.float32),
            pltpu.VMEM_SHARED((16, nacc), jnp.float32),
        ],
    )
    return fn(x_flat, edge_index)


def _tc_body(n, hdim, x_r, sp_r, dp_r, u_r, be_r, ws_r, wn_r,
             bp_r, wd_r, bd_r, wt_r, bt_r, y_r, h_r, t_r):
    f32 = jnp.float32
    dn0 = (((0,), (0,)), ((), ()))
    u = u_r[...]
    be = be_r[...]
    ws = ws_r[...]
    wn = wn_r[...]
    a1 = jnp.dot(u, ws, preferred_element_type=f32)
    a2 = jnp.dot(u, wn, preferred_element_type=f32)
    a3 = jnp.dot(be, wn, preferred_element_type=f32)
    a4 = jnp.dot(be, ws, preferred_element_type=f32) + bp_r[...]
    amat = jnp.concatenate([a1, a2, a3, a4], axis=0)

    x_row = x_r[...].reshape(1, n)
    s_row = sp_r[0:1, :n] + sp_r[1:2, :n]
    deg = dp_r[0:1, :n] + dp_r[1:2, :n]
    cde = jnp.maximum(deg, 1.0)
    r_row = s_row / cde
    m_row = jnp.minimum(deg, 1.0)
    rows4 = jnp.concatenate(
        [x_row, r_row, m_row, jnp.ones((1, n), f32)], axis=0)
    x4 = lax.dot_general(rows4, jnp.eye(4, dtype=f32), dn0,
                         preferred_element_type=f32)

    pre = jnp.dot(x4, amat, preferred_element_type=f32)
    h = jnp.maximum(pre, 0.0)
    h_r[...] = h

    wd = wd_r[...]
    wd1 = wd[:hdim, :]
    wd2 = wd[hdim:, :]
    c1 = jnp.dot(u, wd2, preferred_element_type=f32)
    c0 = jnp.dot(be, wd2, preferred_element_type=f32)
    x_col = x4[:, 0:1]
    logits = (jnp.dot(h, wd1, preferred_element_type=f32)
              + x_col * c1 + c0 + bd_r[...])
    y_r[...] = jax.nn.sigmoid(logits)

    psum = jnp.sum(h, axis=0, keepdims=True)
    pmax = jnp.max(h, axis=0, keepdims=True)
    wt = wt_r[...]
    tt = (jnp.dot(pmax, wt[:hdim, :], preferred_element_type=f32)
          + jnp.dot(psum * (1.0 / n), wt[hdim:, :], preferred_element_type=f32)
          + bt_r[...])
    t_r[...] = jax.nn.sigmoid(tt)


def kernel(x, edge_index, W_enc, b_enc, W_self, W_neigh, b_proc, W_dec,
           b_dec, W_term, b_term):
    n = x.shape[0]
    hdim = W_self.shape[0]
    x_flat = x.reshape(n)

    s_parts, deg_parts = _segment_sums(x_flat, edge_index, n)

    u = W_enc[0:1, :]
    be = b_enc.reshape(1, hdim)
    bp = b_proc.reshape(1, hdim)
    bd = b_dec.reshape(1, 1)
    bt = b_term.reshape(1, 1)

    full = lambda a: pl.BlockSpec(a.shape, lambda: (0,) * a.ndim)
    args = (x_flat, s_parts, deg_parts, u, be, W_self, W_neigh, bp,
            W_dec, bd, W_term, bt)
    y, h, t2 = pl.pallas_call(
        functools.partial(_tc_body, n, hdim),
        in_specs=[full(a) for a in args],
        out_specs=[pl.BlockSpec((n, 1), lambda: (0, 0)),
                   pl.BlockSpec((n, hdim), lambda: (0, 0)),
                   pl.BlockSpec((1, 1), lambda: (0, 0))],
        out_shape=[jax.ShapeDtypeStruct((n, 1), jnp.float32),
                   jax.ShapeDtypeStruct((n, hdim), jnp.float32),
                   jax.ShapeDtypeStruct((1, 1), jnp.float32)],
    )(*args)

    return (y, h, t2.reshape(1))


# speedup vs baseline: 1.0698x; 1.0698x over previous
import functools

import jax
import jax.numpy as jnp
from jax import lax
from jax.experimental import pallas as pl
from jax.experimental.pallas import tpu as pltpu
from jax.experimental.pallas import tpu_sc as plsc

NW = 32
NC = 2


def _sc_body(per_w, nacc, x_hbm, edge_hbm,
             s_out, deg_out, x_v, src_v, dst_v, s_part, deg_part,
             sbuf, dbuf, os_v, od_v, sem, s_all, deg_all):
    c = lax.axis_index("c")
    s = lax.axis_index("s")
    wid = s * NC + c
    cp1 = pltpu.async_copy(x_hbm, x_v, sem)
    cp2 = pltpu.async_copy(edge_hbm.at[0, pl.ds(wid * per_w, per_w)], src_v, sem)
    cp3 = pltpu.async_copy(edge_hbm.at[1, pl.ds(wid * per_w, per_w)], dst_v, sem)

    zeros16 = jnp.zeros((16,), jnp.float32)

    @plsc.parallel_loop(0, nacc // 16, unroll=8)
    def _zero(j):
        sl = pl.ds(j * 16, 16)
        s_part[sl] = zeros16
        deg_part[sl] = zeros16

    cp1.wait()
    cp2.wait()
    cp3.wait()

    ones16 = jnp.ones((16,), jnp.float32)

    def one_vreg(j):
        base = j * 16
        idx_s = src_v[pl.ds(base, 16)]
        idx_d = dst_v[pl.ds(base, 16)]
        vals = plsc.load_gather(x_v, [idx_s])
        plsc.addupdate_scatter(s_part, [idx_d], vals)
        plsc.addupdate_scatter(deg_part, [idx_d], ones16)

    plsc.parallel_loop(0, per_w // 16, unroll=8)(one_vreg)

    pltpu.sync_copy(s_part, s_all.at[s])
    pltpu.sync_copy(deg_part, deg_all.at[s])
    plsc.subcore_barrier()

    w = nacc // 16
    pltpu.sync_copy(s_all.at[:, pl.ds(s * w, w)], sbuf)
    pltpu.sync_copy(deg_all.at[:, pl.ds(s * w, w)], dbuf)

    @plsc.parallel_loop(0, w // 16, unroll=2)
    def _red(v):
        sl = pl.ds(v * 16, 16)
        accs = sbuf[0, sl]
        accd = dbuf[0, sl]
        for k in range(1, 16):
            accs = accs + sbuf[k, sl]
            accd = accd + dbuf[k, sl]
        os_v[sl] = accs
        od_v[sl] = accd
    pltpu.sync_copy(os_v, s_out.at[pl.ds(c * nacc + s * w, w)])
    pltpu.sync_copy(od_v, deg_out.at[pl.ds(c * nacc + s * w, w)])


def _segment_sums(x_flat, edge_index, n):
    e = edge_index.shape[1]
    per_w = e // NW
    nacc = -(-n // 256) * 256
    w = nacc // 16

    fn = pl.kernel(
        functools.partial(_sc_body, per_w, nacc),
        out_type=[jax.ShapeDtypeStruct((NC * nacc,), jnp.float32),
                  jax.ShapeDtypeStruct((NC * nacc,), jnp.float32)],
        mesh=plsc.VectorSubcoreMesh(core_axis_name="c", subcore_axis_name="s"),
        compiler_params=pltpu.CompilerParams(needs_layout_passes=False,
                                             use_tc_tiling_on_sc=False,
                                             skip_device_barrier=True),
        scratch_types=[
            pltpu.VMEM((x_flat.shape[0],), jnp.float32),
            pltpu.VMEM((per_w,), jnp.int32),
            pltpu.VMEM((per_w,), jnp.int32),
            pltpu.VMEM((nacc,), jnp.float32),
            pltpu.VMEM((nacc,), jnp.float32),
            pltpu.VMEM((16, w), jnp.float32),
            pltpu.VMEM((16, w), jnp.float32),
            pltpu.VMEM((w,), jnp.float32),
            pltpu.VMEM((w,), jnp.float32),
            pltpu.SemaphoreType.DMA,
            pltpu.VMEM_SHARED((16, nacc), jnp.float32),
            pltpu.VMEM_SHARED((16, nacc), jnp.float32),
        ],
    )
    return fn(x_flat, edge_index)


def _tc_body(n, hdim, nacc, x_r, sp_r, dp_r, u_r, be_r, ws_r, wn_r,
             bp_r, wd_r, bd_r, wt_r, bt_r, y_r, h_r, t_r):
    f32 = jnp.float32
    dn0 = (((0,), (0,)), ((), ()))
    u = u_r[...]
    be = be_r[...]
    ws = ws_r[...]
    wn = wn_r[...]
    a1 = jnp.dot(u, ws, preferred_element_type=f32)
    a2 = jnp.dot(u, wn, preferred_element_type=f32)
    a3 = jnp.dot(be, wn, preferred_element_type=f32)
    a4 = jnp.dot(be, ws, preferred_element_type=f32) + bp_r[...]
    amat = jnp.concatenate([a1, a2, a3, a4], axis=0)

    x_row = x_r[...].reshape(1, n)
    sp = sp_r[...]
    dp = dp_r[...]
    s_row = (sp[0:n] + sp[nacc:nacc + n]).reshape(1, n)
    deg = (dp[0:n] + dp[nacc:nacc + n]).reshape(1, n)
    cde = jnp.maximum(deg, 1.0)
    r_row = s_row / cde
    m_row = jnp.minimum(deg, 1.0)
    rows4 = jnp.concatenate(
        [x_row, r_row, m_row, jnp.ones((1, n), f32)], axis=0)
    x4 = lax.dot_general(rows4, jnp.eye(4, dtype=f32), dn0,
                         preferred_element_type=f32)

    pre = jnp.dot(x4, amat, preferred_element_type=f32)
    h = jnp.maximum(pre, 0.0)
    h_r[...] = h

    wd = wd_r[...]
    wd1 = wd[:hdim, :]
    wd2 = wd[hdim:, :]
    c1 = jnp.dot(u, wd2, preferred_element_type=f32)
    c0 = jnp.dot(be, wd2, preferred_element_type=f32)
    x_col = x4[:, 0:1]
    logits = (jnp.dot(h, wd1, preferred_element_type=f32)
              + x_col * c1 + c0 + bd_r[...])
    y_r[...] = jax.nn.sigmoid(logits)

    psum = jnp.sum(h, axis=0, keepdims=True)
    pmax = jnp.max(h, axis=0, keepdims=True)
    wt = wt_r[...]
    tt = (jnp.dot(pmax, wt[:hdim, :], preferred_element_type=f32)
          + jnp.dot(psum * (1.0 / n), wt[hdim:, :], preferred_element_type=f32)
          + bt_r[...])
    t_r[...] = jax.nn.sigmoid(tt)


def kernel(x, edge_index, W_enc, b_enc, W_self, W_neigh, b_proc, W_dec,
           b_dec, W_term, b_term):
    n = x.shape[0]
    hdim = W_self.shape[0]
    x_flat = x.reshape(n)

    s_parts, deg_parts = _segment_sums(x_flat, edge_index, n)

    u = W_enc[0:1, :]
    be = b_enc.reshape(1, hdim)
    bp = b_proc.reshape(1, hdim)
    bd = b_dec.reshape(1, 1)
    bt = b_term.reshape(1, 1)

    full = lambda a: pl.BlockSpec(a.shape, lambda: (0,) * a.ndim)
    args = (x_flat, s_parts, deg_parts, u, be, W_self, W_neigh, bp,
            W_dec, bd, W_term, bt)
    nacc = s_parts.shape[0] // NC
    y, h, t2 = pl.pallas_call(
        functools.partial(_tc_body, n, hdim, nacc),
        in_specs=[full(a) for a in args],
        out_specs=[pl.BlockSpec((n, 1), lambda: (0, 0)),
                   pl.BlockSpec((n, hdim), lambda: (0, 0)),
                   pl.BlockSpec((1, 1), lambda: (0, 0))],
        out_shape=[jax.ShapeDtypeStruct((n, 1), jnp.float32),
                   jax.ShapeDtypeStruct((n, hdim), jnp.float32),
                   jax.ShapeDtypeStruct((1, 1), jnp.float32)],
    )(*args)

    return (y, h, t2.reshape(1))


# speedup vs baseline: 1.1375x; 1.0633x over previous
import functools

import jax
import jax.numpy as jnp
from jax import lax
from jax.experimental import pallas as pl
from jax.experimental.pallas import tpu as pltpu
from jax.experimental.pallas import tpu_sc as plsc

NW = 32
NC = 2


def _sc_body(main_w, rem, nacc, x_hbm, edge_hbm,
             s_out, deg_out, x_v, ev_v, ex_v, s_part, deg_part,
             sbuf, dbuf, os_v, od_v, sem, s_all, deg_all):
    c = lax.axis_index("c")
    s = lax.axis_index("s")
    wid = s * NC + c
    cp1 = pltpu.async_copy(x_hbm, x_v, sem)
    cp2 = pltpu.async_copy(edge_hbm.at[:, pl.ds(wid * main_w, main_w)], ev_v,
                           sem)

    @pl.when(wid < rem)
    def _():
        pltpu.sync_copy(
            edge_hbm.at[:, pl.ds(NW * main_w + wid * 128, 128)], ex_v)

    zeros16 = jnp.zeros((16,), jnp.float32)

    @plsc.parallel_loop(0, nacc // 16, unroll=8)
    def _zero(j):
        sl = pl.ds(j * 16, 16)
        s_part[sl] = zeros16
        deg_part[sl] = zeros16

    cp1.wait()
    cp2.wait()

    ones16 = jnp.ones((16,), jnp.float32)

    def edge_vreg(ref, base):
        idx_s = ref[0, pl.ds(base, 16)]
        idx_d = ref[1, pl.ds(base, 16)]
        vals = plsc.load_gather(x_v, [idx_s])
        plsc.addupdate_scatter(s_part, [idx_d], vals)
        plsc.addupdate_scatter(deg_part, [idx_d], ones16)

    @plsc.parallel_loop(0, main_w // 16, unroll=8)
    def _edges(j):
        edge_vreg(ev_v, j * 16)

    @pl.when(wid < rem)
    def _():
        for k in range(128 // 16):
            edge_vreg(ex_v, k * 16)

    pltpu.sync_copy(s_part, s_all.at[s])
    pltpu.sync_copy(deg_part, deg_all.at[s])
    plsc.subcore_barrier()

    w = nacc // 16
    pltpu.sync_copy(s_all.at[:, pl.ds(s * w, w)], sbuf)
    pltpu.sync_copy(deg_all.at[:, pl.ds(s * w, w)], dbuf)

    @plsc.parallel_loop(0, w // 16, unroll=2)
    def _red(v):
        sl = pl.ds(v * 16, 16)
        accs = sbuf[0, sl]
        accd = dbuf[0, sl]
        for k in range(1, 16):
            accs = accs + sbuf[k, sl]
            accd = accd + dbuf[k, sl]
        os_v[sl] = accs
        od_v[sl] = accd
    pltpu.sync_copy(os_v, s_out.at[pl.ds(c * nacc + s * w, w)])
    pltpu.sync_copy(od_v, deg_out.at[pl.ds(c * nacc + s * w, w)])


def _segment_sums(x_flat, edge_index, n):
    e = edge_index.shape[1]
    chunks = e // 128
    base_ch = chunks // NW
    rem = chunks - base_ch * NW
    main_w = base_ch * 128
    nacc = -(-n // 256) * 256
    w = nacc // 16

    fn = pl.kernel(
        functools.partial(_sc_body, main_w, rem, nacc),
        out_type=[jax.ShapeDtypeStruct((NC * nacc,), jnp.float32),
                  jax.ShapeDtypeStruct((NC * nacc,), jnp.float32)],
        mesh=plsc.VectorSubcoreMesh(core_axis_name="c", subcore_axis_name="s"),
        compiler_params=pltpu.CompilerParams(needs_layout_passes=False,
                                             skip_device_barrier=True),
        scratch_types=[
            pltpu.VMEM((x_flat.shape[0],), jnp.float32),
            pltpu.VMEM((2, main_w), jnp.int32),
            pltpu.VMEM((2, 128), jnp.int32),
            pltpu.VMEM((nacc,), jnp.float32),
            pltpu.VMEM((nacc,), jnp.float32),
            pltpu.VMEM((16, w), jnp.float32),
            pltpu.VMEM((16, w), jnp.float32),
            pltpu.VMEM((w,), jnp.float32),
            pltpu.VMEM((w,), jnp.float32),
            pltpu.SemaphoreType.DMA,
            pltpu.VMEM_SHARED((16, nacc), jnp.float32),
            pltpu.VMEM_SHARED((16, nacc), jnp.float32),
        ],
    )
    return fn(x_flat, edge_index)


def _tc_body(n, hdim, nacc, x_r, sp_r, dp_r, u_r, be_r, ws_r, wn_r,
             bp_r, wd_r, bd_r, wt_r, bt_r, y_r, h_r, t_r):
    f32 = jnp.float32
    dn0 = (((0,), (0,)), ((), ()))
    u = u_r[...]
    be = be_r[...]
    ws = ws_r[...]
    wn = wn_r[...]
    a1 = jnp.dot(u, ws, preferred_element_type=f32)
    a2 = jnp.dot(u, wn, preferred_element_type=f32)
    a3 = jnp.dot(be, wn, preferred_element_type=f32)
    a4 = jnp.dot(be, ws, preferred_element_type=f32) + bp_r[...]
    amat = jnp.concatenate([a1, a2, a3, a4], axis=0)

    x_row = x_r[...].reshape(1, n)
    sp = sp_r[...]
    dp = dp_r[...]
    s_row = (sp[0:n] + sp[nacc:nacc + n]).reshape(1, n)
    deg = (dp[0:n] + dp[nacc:nacc + n]).reshape(1, n)
    cde = jnp.maximum(deg, 1.0)
    r_row = s_row / cde
    m_row = jnp.minimum(deg, 1.0)
    rows4 = jnp.concatenate(
        [x_row, r_row, m_row, jnp.ones((1, n), f32)], axis=0)
    x4 = lax.dot_general(rows4, jnp.eye(4, dtype=f32), dn0,
                         preferred_element_type=f32)

    pre = jnp.dot(x4, amat, preferred_element_type=f32)
    h = jnp.maximum(pre, 0.0)
    h_r[...] = h

    wd = wd_r[...]
    wd1 = wd[:hdim, :]
    wd2 = wd[hdim:, :]
    c1 = jnp.dot(u, wd2, preferred_element_type=f32)
    c0 = jnp.dot(be, wd2, preferred_element_type=f32)
    x_col = x4[:, 0:1]
    logits = (jnp.dot(h, wd1, preferred_element_type=f32)
              + x_col * c1 + c0 + bd_r[...])
    y_r[...] = jax.nn.sigmoid(logits)

    psum = jnp.sum(h, axis=0, keepdims=True)
    pmax = jnp.max(h, axis=0, keepdims=True)
    wt = wt_r[...]
    tt = (jnp.dot(pmax, wt[:hdim, :], preferred_element_type=f32)
          + jnp.dot(psum * (1.0 / n), wt[hdim:, :], preferred_element_type=f32)
          + bt_r[...])
    t_r[...] = jax.nn.sigmoid(tt)


def kernel(x, edge_index, W_enc, b_enc, W_self, W_neigh, b_proc, W_dec,
           b_dec, W_term, b_term):
    n = x.shape[0]
    hdim = W_self.shape[0]
    x_flat = x.reshape(n)

    s_parts, deg_parts = _segment_sums(x_flat, edge_index, n)

    u = W_enc[0:1, :]
    be = b_enc.reshape(1, hdim)
    bp = b_proc.reshape(1, hdim)
    bd = b_dec.reshape(1, 1)
    bt = b_term.reshape(1, 1)

    full = lambda a: pl.BlockSpec(a.shape, lambda: (0,) * a.ndim)
    args = (x_flat, s_parts, deg_parts, u, be, W_self, W_neigh, bp,
            W_dec, bd, W_term, bt)
    nacc = s_parts.shape[0] // NC
    y, h, t2 = pl.pallas_call(
        functools.partial(_tc_body, n, hdim, nacc),
        in_specs=[full(a) for a in args],
        out_specs=[pl.BlockSpec((n, 1), lambda: (0, 0)),
                   pl.BlockSpec((n, hdim), lambda: (0, 0)),
                   pl.BlockSpec((1, 1), lambda: (0, 0))],
        out_shape=[jax.ShapeDtypeStruct((n, 1), jnp.float32),
                   jax.ShapeDtypeStruct((n, hdim), jnp.float32),
                   jax.ShapeDtypeStruct((1, 1), jnp.float32)],
    )(*args)

    return (y, h, t2.reshape(1))
